# Initial kernel scaffold; baseline (speedup 1.0000x reference)
import jax
import jax.numpy as jnp
from jax.experimental import pallas as pl


def kernel(target_series, TS_database):
    raise NotImplementedError("write your pallas kernel here")


# speedup vs baseline: 1.4481x; 1.4481x over previous
import functools

import numpy as np
import jax
import jax.numpy as jnp
from jax.experimental import pallas as pl
from jax.experimental.pallas import tpu as pltpu

_NPERSEG = 128
_STEP = 64
_NSEG = 7
_NF = 65
_NREF = 64
_BLK = 512


def _dft_mats():
    n = _NPERSEG
    nn = np.arange(n, dtype=np.float64)
    win = 0.5 - 0.5 * np.cos(2.0 * np.pi * nn / n)
    f = np.arange(_NF, dtype=np.float64)
    ang = 2.0 * np.pi * np.outer(nn, f) / n
    cr = np.cos(ang)
    ci = -np.sin(ang)
    wc = np.zeros((n, n))
    wc[:, :_NF] = cr
    wc[:, _NF:] = ci[:, 1:_NF - 1]
    wsw = np.zeros((n, n))
    wsw[:, :_NF] = ci
    wsw[:, _NF:] = -cr[:, 1:_NF - 1]
    wc = win[:, None] * wc
    wsw = win[:, None] * wsw
    return (jnp.asarray(wc, jnp.float32), jnp.asarray(wsw, jnp.float32))


def _scores_kernel(t_ref, db_ref, wc_ref, wsw_ref, out_ref):
    wc = wc_ref[...]
    wsw = wsw_ref[...]
    hi = jax.lax.Precision.HIGHEST
    dn = (((1,), (0,)), ((), ()))
    p2 = jnp.zeros((_BLK, 128), jnp.float32)
    m = jnp.zeros((_BLK, 128), jnp.float32)
    n2 = jnp.zeros((_BLK, 128), jnp.float32)
    pxx2 = jnp.zeros((1, 128), jnp.float32)
    for s in range(_NSEG):
        sl = slice(_STEP * s, _STEP * s + _NPERSEG)
        d = db_ref[:, sl]
        y = jax.lax.dot_general(d, wc, dn, precision=hi,
                                preferred_element_type=jnp.float32)
        t = t_ref[0, :, sl]
        x = jax.lax.dot_general(t, wc, dn, precision=hi,
                                preferred_element_type=jnp.float32)
        xs = jax.lax.dot_general(t, wsw, dn, precision=hi,
                                 preferred_element_type=jnp.float32)
        p2 = p2 + y * y
        m = m + x * y
        n2 = n2 + xs * y
        pxx2 = pxx2 + x * x

    fidx = jax.lax.broadcasted_iota(jnp.int32, (1, 128), 1)
    mid = jnp.logical_and(fidx >= 1, fidx <= 63)

    def fold(a):
        a = a * (1.0 / _NSEG)
        rolled = jnp.concatenate([a[:, _STEP:], a[:, :_STEP]], axis=1)
        return a + jnp.where(mid, rolled, 0.0)

    pyy = fold(p2)
    pr = fold(m)
    pi = fold(n2)
    pxx = fold(pxx2)
    cxy = (pr * pr + pi * pi) / (pxx * pyy + 1e-12)
    mask65 = fidx <= 64
    score = jnp.sum(jnp.where(mask65, cxy, 0.0), axis=1, keepdims=True)
    out_ref[...] = (score * (1.0 / _NF)).reshape(1, _BLK, 1)


def _topk_kernel(s_ref, idx_ref):
    scores = s_ref[...]
    b, n = scores.shape
    iota = jax.lax.broadcasted_iota(jnp.int32, (b, n), 1)
    cols = []
    for _ in range(_NREF):
        mx = jnp.max(scores, axis=1, keepdims=True)
        hit = scores == mx
        idx = jnp.min(jnp.where(hit, iota, jnp.int32(2**30)), axis=1,
                      keepdims=True)
        cols.append(idx)
        scores = jnp.where(iota == idx, jnp.float32(-1.0), scores)
    idx_ref[...] = jnp.concatenate(cols, axis=1)


def _gather_kernel(idx_ref, db_ref, out_ref):
    del idx_ref
    out_ref[...] = db_ref[...]


def kernel(target_series, TS_database):
    B, N, L = TS_database.shape
    nblk = (B * N) // _BLK
    wc, wsw = _dft_mats()
    db_flat = TS_database.reshape(B * N, L)
    blocks_per_batch = N // _BLK

    scores3 = pl.pallas_call(
        _scores_kernel,
        grid=(nblk,),
        in_specs=[
            pl.BlockSpec((1, 1, L), lambda j: (j // blocks_per_batch, 0, 0)),
            pl.BlockSpec((_BLK, L), lambda j: (j, 0)),
            pl.BlockSpec((128, 128), lambda j: (0, 0)),
            pl.BlockSpec((128, 128), lambda j: (0, 0)),
        ],
        out_specs=pl.BlockSpec((1, _BLK, 1), lambda j: (j, 0, 0)),
        out_shape=jax.ShapeDtypeStruct((nblk, _BLK, 1), jnp.float32),
        compiler_params=pltpu.CompilerParams(
            dimension_semantics=("arbitrary",)),
    )(target_series, db_flat, wc, wsw)
    scores = scores3.reshape(B, N)

    topk_idx = pl.pallas_call(
        _topk_kernel,
        out_shape=jax.ShapeDtypeStruct((B, _NREF), jnp.int32),
    )(scores)

    db4 = TS_database.reshape(B, N, L // 128, 128)
    gathered = pl.pallas_call(
        _gather_kernel,
        grid_spec=pltpu.PrefetchScalarGridSpec(
            num_scalar_prefetch=1,
            grid=(B, _NREF),
            in_specs=[
                pl.BlockSpec((1, 1, L // 128, 128),
                             lambda b, k, idx: (b, idx[b, k], 0, 0)),
            ],
            out_specs=pl.BlockSpec((1, 1, L // 128, 128),
                                   lambda b, k, idx: (b, k, 0, 0)),
        ),
        out_shape=jax.ShapeDtypeStruct((B, _NREF, L // 128, 128), jnp.float32),
    )(topk_idx, db4)
    return gathered.reshape(B, _NREF, L)


# speedup vs baseline: 3.1758x; 2.1930x over previous
import functools

import numpy as np
import jax
import jax.numpy as jnp
from jax import lax
from jax.experimental import pallas as pl
from jax.experimental.pallas import tpu as pltpu
from jax.experimental.pallas import tpu_sc as plsc

_NPERSEG = 128
_STEP = 64
_NSEG = 7
_NF = 65
_NREF = 64
_BLK = 512


def _dft_mats():
    n = _NPERSEG
    nn = np.arange(n, dtype=np.float64)
    win = 0.5 - 0.5 * np.cos(2.0 * np.pi * nn / n)
    f = np.arange(_NF, dtype=np.float64)
    ang = 2.0 * np.pi * np.outer(nn, f) / n
    cr = np.cos(ang)
    ci = -np.sin(ang)
    wc = np.zeros((n, n))
    wc[:, :_NF] = cr
    wc[:, _NF:] = ci[:, 1:_NF - 1]
    wsw = np.zeros((n, n))
    wsw[:, :_NF] = ci
    wsw[:, _NF:] = -cr[:, 1:_NF - 1]
    wc = win[:, None] * wc
    wsw = win[:, None] * wsw
    return (jnp.asarray(wc, jnp.float32), jnp.asarray(wsw, jnp.float32))


def _scores_kernel(t_ref, db_ref, wc_ref, wsw_ref, out_ref):
    wc = wc_ref[...]
    wsw = wsw_ref[...]
    hi = jax.lax.Precision.HIGHEST
    dn = (((1,), (0,)), ((), ()))
    p2 = jnp.zeros((_BLK, 128), jnp.float32)
    m = jnp.zeros((_BLK, 128), jnp.float32)
    n2 = jnp.zeros((_BLK, 128), jnp.float32)
    pxx2 = jnp.zeros((1, 128), jnp.float32)
    for s in range(_NSEG):
        sl = slice(_STEP * s, _STEP * s + _NPERSEG)
        d = db_ref[:, sl]
        y = jax.lax.dot_general(d, wc, dn, precision=hi,
                                preferred_element_type=jnp.float32)
        t = t_ref[0, :, sl]
        x = jax.lax.dot_general(t, wc, dn, precision=hi,
                                preferred_element_type=jnp.float32)
        xs = jax.lax.dot_general(t, wsw, dn, precision=hi,
                                 preferred_element_type=jnp.float32)
        p2 = p2 + y * y
        m = m + x * y
        n2 = n2 + xs * y
        pxx2 = pxx2 + x * x

    fidx = jax.lax.broadcasted_iota(jnp.int32, (1, 128), 1)
    mid = jnp.logical_and(fidx >= 1, fidx <= 63)

    def fold(a):
        a = a * (1.0 / _NSEG)
        rolled = jnp.concatenate([a[:, _STEP:], a[:, :_STEP]], axis=1)
        return a + jnp.where(mid, rolled, 0.0)

    pyy = fold(p2)
    pr = fold(m)
    pi = fold(n2)
    pxx = fold(pxx2)
    cxy = (pr * pr + pi * pi) / (pxx * pyy + 1e-12)
    mask65 = fidx <= 64
    score = jnp.sum(jnp.where(mask65, cxy, 0.0), axis=1, keepdims=True)
    out_ref[...] = (score * (1.0 / _NF)).reshape(1, _BLK, 1)


def _topk_kernel(s_ref, idx_ref):
    scores = s_ref[...]
    b, n = scores.shape
    iota = jax.lax.broadcasted_iota(jnp.int32, (b, n), 1)
    base = jax.lax.broadcasted_iota(jnp.int32, (b, 1), 0) * n
    cols = []
    for _ in range(_NREF):
        mx = jnp.max(scores, axis=1, keepdims=True)
        hit = scores == mx
        idx = jnp.min(jnp.where(hit, iota, jnp.int32(2**30)), axis=1,
                      keepdims=True)
        cols.append(idx + base)
        scores = jnp.where(iota == idx, jnp.float32(-1.0), scores)
    idx_ref[...] = jnp.concatenate(cols, axis=1)


def _sc_gather(db_flat, flat_idx, n_rows, L):
    info = plsc.get_sparse_core_info()
    nw = info.num_cores * info.num_subcores
    rows_per_w = n_rows // nw
    mesh = plsc.VectorSubcoreMesh(core_axis_name="c", subcore_axis_name="s")

    @functools.partial(
        pl.kernel, mesh=mesh,
        out_type=jax.ShapeDtypeStruct((n_rows, L), jnp.float32),
        scratch_types=[
            pltpu.VMEM((rows_per_w,), jnp.int32),
            pltpu.VMEM((rows_per_w, L), jnp.float32),
            pltpu.SemaphoreType.DMA,
        ],
    )
    def gather(db_hbm, idx_hbm, out_hbm, idx_v, rows_v, sem):
        wid = lax.axis_index("s") * info.num_cores + lax.axis_index("c")
        base = wid * rows_per_w
        pltpu.sync_copy(idx_hbm.at[pl.ds(base, rows_per_w)], idx_v)
        pltpu.async_copy(db_hbm.at[idx_v], rows_v, sem).wait()
        pltpu.sync_copy(rows_v, out_hbm.at[pl.ds(base, rows_per_w)])

    return gather(db_flat, flat_idx)


def kernel(target_series, TS_database):
    B, N, L = TS_database.shape
    nblk = (B * N) // _BLK
    wc, wsw = _dft_mats()
    db_flat = TS_database.reshape(B * N, L)
    blocks_per_batch = N // _BLK

    scores3 = pl.pallas_call(
        _scores_kernel,
        grid=(nblk,),
        in_specs=[
            pl.BlockSpec((1, 1, L), lambda j: (j // blocks_per_batch, 0, 0)),
            pl.BlockSpec((_BLK, L), lambda j: (j, 0)),
            pl.BlockSpec((128, 128), lambda j: (0, 0)),
            pl.BlockSpec((128, 128), lambda j: (0, 0)),
        ],
        out_specs=pl.BlockSpec((1, _BLK, 1), lambda j: (j, 0, 0)),
        out_shape=jax.ShapeDtypeStruct((nblk, _BLK, 1), jnp.float32),
        compiler_params=pltpu.CompilerParams(
            dimension_semantics=("arbitrary",)),
    )(target_series, db_flat, wc, wsw)
    scores = scores3.reshape(B, N)

    topk_idx = pl.pallas_call(
        _topk_kernel,
        out_shape=jax.ShapeDtypeStruct((B, _NREF), jnp.int32),
    )(scores)

    gathered = _sc_gather(db_flat, topk_idx.reshape(B * _NREF), B * _NREF, L)
    return gathered.reshape(B, _NREF, L)


# speedup vs baseline: 4.2773x; 1.3468x over previous
import functools

import numpy as np
import jax
import jax.numpy as jnp
from jax import lax
from jax.experimental import pallas as pl
from jax.experimental.pallas import tpu as pltpu
from jax.experimental.pallas import tpu_sc as plsc

_NPERSEG = 128
_STEP = 64
_NSEG = 7
_NF = 65
_NREF = 64
_M1 = 128
_BLK = 512


def _dft_mats():
    n = _NPERSEG
    nn = np.arange(n, dtype=np.float64)
    win = 0.5 - 0.5 * np.cos(2.0 * np.pi * nn / n)
    f = np.arange(_NF, dtype=np.float64)
    ang = 2.0 * np.pi * np.outer(nn, f) / n
    cr = np.cos(ang)
    ci = -np.sin(ang)
    wc = np.zeros((n, n))
    wc[:, :_NF] = cr
    wc[:, _NF:] = ci[:, 1:_NF - 1]
    wsw = np.zeros((n, n))
    wsw[:, :_NF] = ci
    wsw[:, _NF:] = -cr[:, 1:_NF - 1]
    wc = win[:, None] * wc
    wsw = win[:, None] * wsw
    return (jnp.asarray(wc, jnp.float32), jnp.asarray(wsw, jnp.float32))


def _fold(a, mid):
    a = a * (1.0 / _NSEG)
    rolled = jnp.concatenate([a[:, _STEP:], a[:, :_STEP]], axis=1)
    return a + jnp.where(mid, rolled, 0.0)


def _stage1_kernel(t_ref, db_ref, wc_ref, wsw_ref, out_ref):
    wc = wc_ref[...]
    wsw = wsw_ref[...]
    dn = (((1,), (0,)), ((), ()))
    p2 = jnp.zeros((_BLK, 128), jnp.float32)
    m = jnp.zeros((_BLK, 128), jnp.float32)
    n2 = jnp.zeros((_BLK, 128), jnp.float32)
    pxx2 = jnp.zeros((1, 128), jnp.float32)
    for s in range(_NSEG):
        sl = slice(_STEP * s, _STEP * s + _NPERSEG)
        d = db_ref[:, sl].astype(jnp.bfloat16)
        y = jax.lax.dot_general(d, wc, dn,
                                preferred_element_type=jnp.float32)
        t = t_ref[0, :, sl].astype(jnp.bfloat16)
        x = jax.lax.dot_general(t, wc, dn,
                                preferred_element_type=jnp.float32)
        xs = jax.lax.dot_general(t, wsw, dn,
                                 preferred_element_type=jnp.float32)
        p2 = p2 + y * y
        m = m + x * y
        n2 = n2 + xs * y
        pxx2 = pxx2 + x * x
    fidx = jax.lax.broadcasted_iota(jnp.int32, (1, 128), 1)
    mid = jnp.logical_and(fidx >= 1, fidx <= 63)
    pyy = _fold(p2, mid)
    pr = _fold(m, mid)
    pi = _fold(n2, mid)
    pxx = _fold(pxx2, mid)
    cxy = (pr * pr + pi * pi) / (pxx * pyy + 1e-12)
    score = jnp.sum(jnp.where(fidx <= 64, cxy, 0.0), axis=1, keepdims=True)
    out_ref[...] = (score * (1.0 / _NF)).reshape(1, _BLK, 1)


def _topk_kernel(s_ref, idx_ref):
    scores = s_ref[...]
    b, n = scores.shape
    iota = jax.lax.broadcasted_iota(jnp.int32, (b, n), 1)
    base = jax.lax.broadcasted_iota(jnp.int32, (b, 1), 0) * n
    cols = []
    for _ in range(_M1):
        mx = jnp.max(scores, axis=1, keepdims=True)
        hit = scores == mx
        idx = jnp.min(jnp.where(hit, iota, jnp.int32(2**30)), axis=1,
                      keepdims=True)
        cols.append(idx + base)
        scores = jnp.where(iota == idx, jnp.float32(-1.0), scores)
    idx_ref[...] = jnp.concatenate(cols, axis=1)


def _rescore_kernel(t_ref, rows_ref, gidx_ref, wc_ref, wsw_ref, out_ref):
    wc = wc_ref[...]
    wsw = wsw_ref[...]
    hi = jax.lax.Precision.HIGHEST
    dn = (((1,), (0,)), ((), ()))
    nb = t_ref.shape[0]
    rows_n = nb * _M1
    p2 = jnp.zeros((rows_n, 128), jnp.float32)
    m = jnp.zeros((rows_n, 128), jnp.float32)
    n2 = jnp.zeros((rows_n, 128), jnp.float32)
    pxx2 = jnp.zeros((nb, 128), jnp.float32)

    def batch_rows(x):
        return jnp.reshape(
            jnp.broadcast_to(x[:, None, :], (nb, _M1, 128)), (rows_n, 128))

    for s in range(_NSEG):
        sl = slice(_STEP * s, _STEP * s + _NPERSEG)
        d = rows_ref[:, sl]
        y = jax.lax.dot_general(d, wc, dn, precision=hi,
                                preferred_element_type=jnp.float32)
        t = t_ref[:, 0, sl]
        x = jax.lax.dot_general(t, wc, dn, precision=hi,
                                preferred_element_type=jnp.float32)
        xs = jax.lax.dot_general(t, wsw, dn, precision=hi,
                                 preferred_element_type=jnp.float32)
        p2 = p2 + y * y
        m = m + batch_rows(x) * y
        n2 = n2 + batch_rows(xs) * y
        pxx2 = pxx2 + x * x
    fidx = jax.lax.broadcasted_iota(jnp.int32, (1, 128), 1)
    mid = jnp.logical_and(fidx >= 1, fidx <= 63)
    pyy = _fold(p2, mid)
    pr = _fold(m, mid)
    pi = _fold(n2, mid)
    pxx = batch_rows(_fold(pxx2, mid))
    cxy = (pr * pr + pi * pi) / (pxx * pyy + 1e-12)
    score = jnp.sum(jnp.where(fidx <= 64, cxy, 0.0), axis=1) * (1.0 / _NF)

    s2 = jnp.reshape(score, (nb, _M1))
    gidx = gidx_ref[...]
    cols = []
    for _ in range(_NREF):
        mx = jnp.max(s2, axis=1, keepdims=True)
        hit = s2 == mx
        g = jnp.min(jnp.where(hit, gidx, jnp.int32(2**30)), axis=1,
                    keepdims=True)
        cols.append(g)
        s2 = jnp.where(gidx == g, jnp.float32(-1.0), s2)
    out_ref[...] = jnp.concatenate(cols, axis=1)


def _sc_gather(db_flat, flat_idx, n_rows, L):
    info = plsc.get_sparse_core_info()
    nw = info.num_cores * info.num_subcores
    rows_per_w = n_rows // nw
    mesh = plsc.VectorSubcoreMesh(core_axis_name="c", subcore_axis_name="s")

    @functools.partial(
        pl.kernel, mesh=mesh,
        out_type=jax.ShapeDtypeStruct((n_rows, L), jnp.float32),
        scratch_types=[
            pltpu.VMEM((rows_per_w,), jnp.int32),
            pltpu.VMEM((rows_per_w, L), jnp.float32),
            pltpu.SemaphoreType.DMA,
        ],
    )
    def gather(db_hbm, idx_hbm, out_hbm, idx_v, rows_v, sem):
        wid = lax.axis_index("s") * info.num_cores + lax.axis_index("c")
        base = wid * rows_per_w
        pltpu.sync_copy(idx_hbm.at[pl.ds(base, rows_per_w)], idx_v)
        pltpu.async_copy(db_hbm.at[idx_v], rows_v, sem).wait()
        pltpu.sync_copy(rows_v, out_hbm.at[pl.ds(base, rows_per_w)])

    return gather(db_flat, flat_idx)


def kernel(target_series, TS_database):
    B, N, L = TS_database.shape
    nblk = (B * N) // _BLK
    wc, wsw = _dft_mats()
    wc_b, wsw_b = wc.astype(jnp.bfloat16), wsw.astype(jnp.bfloat16)
    db_flat = TS_database.reshape(B * N, L)
    blocks_per_batch = N // _BLK

    scores3 = pl.pallas_call(
        _stage1_kernel,
        grid=(nblk,),
        in_specs=[
            pl.BlockSpec((1, 1, L), lambda j: (j // blocks_per_batch, 0, 0)),
            pl.BlockSpec((_BLK, L), lambda j: (j, 0)),
            pl.BlockSpec((128, 128), lambda j: (0, 0)),
            pl.BlockSpec((128, 128), lambda j: (0, 0)),
        ],
        out_specs=pl.BlockSpec((1, _BLK, 1), lambda j: (j, 0, 0)),
        out_shape=jax.ShapeDtypeStruct((nblk, _BLK, 1), jnp.float32),
        compiler_params=pltpu.CompilerParams(
            dimension_semantics=("arbitrary",)),
    )(target_series, db_flat, wc_b, wsw_b)
    scores = scores3.reshape(B, N)

    cand_idx = pl.pallas_call(
        _topk_kernel,
        out_shape=jax.ShapeDtypeStruct((B, _M1), jnp.int32),
    )(scores)

    cand_rows = _sc_gather(db_flat, cand_idx.reshape(B * _M1), B * _M1, L)

    topk_idx = pl.pallas_call(
        _rescore_kernel,
        out_shape=jax.ShapeDtypeStruct((B, _NREF), jnp.int32),
    )(target_series, cand_rows, cand_idx, wc, wsw)

    gathered = _sc_gather(db_flat, topk_idx.reshape(B * _NREF), B * _NREF, L)
    return gathered.reshape(B, _NREF, L)


# speedup vs baseline: 4.5143x; 1.0554x over previous
import functools

import numpy as np
import jax
import jax.numpy as jnp
from jax import lax
from jax.experimental import pallas as pl
from jax.experimental.pallas import tpu as pltpu
from jax.experimental.pallas import tpu_sc as plsc

_NPERSEG = 128
_STEP = 64
_NSEG = 7
_NF = 65
_NREF = 64
_M1 = 128
_BLK = 512


def _dft_mats():
    n = _NPERSEG
    nn = np.arange(n, dtype=np.float64)
    win = 0.5 - 0.5 * np.cos(2.0 * np.pi * nn / n)
    f = np.arange(_NF, dtype=np.float64)
    ang = 2.0 * np.pi * np.outer(nn, f) / n
    cr = np.cos(ang)
    ci = -np.sin(ang)
    wc = np.zeros((n, n))
    wc[:, :_NF] = cr
    wc[:, _NF:] = ci[:, 1:_NF - 1]
    wsw = np.zeros((n, n))
    wsw[:, :_NF] = ci
    wsw[:, _NF:] = -cr[:, 1:_NF - 1]
    wc = win[:, None] * wc
    wsw = win[:, None] * wsw
    return (jnp.asarray(wc, jnp.float32), jnp.asarray(wsw, jnp.float32))


def _fold(a, mid):
    a = a * (1.0 / _NSEG)
    rolled = jnp.concatenate([a[:, _STEP:], a[:, :_STEP]], axis=1)
    return a + jnp.where(mid, rolled, 0.0)


_CHUNK = 128


def _stage1_kernel(t_ref, db_ref, wc_ref, wsw_ref, out_ref):
    wc = wc_ref[...]
    wsw = wsw_ref[...]
    dn = (((1,), (0,)), ((), ()))
    fidx = jax.lax.broadcasted_iota(jnp.int32, (1, 128), 1)
    mid = jnp.logical_and(fidx >= 1, fidx <= 63)

    xs_list = []
    pxx2 = jnp.zeros((1, 128), jnp.float32)
    for s in range(_NSEG):
        sl = slice(_STEP * s, _STEP * s + _NPERSEG)
        t = t_ref[0, :, sl].astype(jnp.bfloat16)
        x = jax.lax.dot_general(t, wc, dn,
                                preferred_element_type=jnp.float32)
        xw = jax.lax.dot_general(t, wsw, dn,
                                 preferred_element_type=jnp.float32)
        xs_list.append((x, xw))
        pxx2 = pxx2 + x * x
    pxx = _fold(pxx2, mid)

    for c in range(_BLK // _CHUNK):
        rows = slice(_CHUNK * c, _CHUNK * (c + 1))
        p2 = jnp.zeros((_CHUNK, 128), jnp.float32)
        m = jnp.zeros((_CHUNK, 128), jnp.float32)
        n2 = jnp.zeros((_CHUNK, 128), jnp.float32)
        for s in range(_NSEG):
            sl = slice(_STEP * s, _STEP * s + _NPERSEG)
            d = db_ref[rows, sl].astype(jnp.bfloat16)
            y = jax.lax.dot_general(d, wc, dn,
                                    preferred_element_type=jnp.float32)
            x, xw = xs_list[s]
            p2 = p2 + y * y
            m = m + x * y
            n2 = n2 + xw * y
        pyy = _fold(p2, mid)
        pr = _fold(m, mid)
        pi = _fold(n2, mid)
        cxy = (pr * pr + pi * pi) / (pxx * pyy + 1e-12)
        score = jnp.sum(jnp.where(fidx <= 64, cxy, 0.0), axis=1,
                        keepdims=True)
        out_ref[0, rows, :] = score


def _topk_kernel(s_ref, idx_ref):
    scores = s_ref[...]
    b, n = scores.shape
    iota = jax.lax.broadcasted_iota(jnp.int32, (b, n), 1)
    base = jax.lax.broadcasted_iota(jnp.int32, (b, 1), 0) * n
    bits = jax.lax.bitcast_convert_type(scores, jnp.int32)
    packed = jnp.bitwise_or(
        jnp.bitwise_and(bits, jnp.int32(~0xFFF)), (n - 1) - iota)
    cols = []
    for _ in range(_M1):
        mx = jnp.max(packed, axis=1, keepdims=True)
        cols.append(((n - 1) - jnp.bitwise_and(mx, jnp.int32(0xFFF))) + base)
        packed = jnp.where(packed == mx, jnp.int32(-2**31), packed)
    idx_ref[...] = jnp.concatenate(cols, axis=1)


def _rescore_kernel(t_ref, rows_ref, gidx_ref, wc_ref, wsw_ref, out_ref):
    wc = wc_ref[...]
    wsw = wsw_ref[...]
    hi = jax.lax.Precision.HIGHEST
    dn = (((1,), (0,)), ((), ()))
    nb = t_ref.shape[0]
    rows_n = nb * _M1
    p2 = jnp.zeros((rows_n, 128), jnp.float32)
    m = jnp.zeros((rows_n, 128), jnp.float32)
    n2 = jnp.zeros((rows_n, 128), jnp.float32)
    pxx2 = jnp.zeros((nb, 128), jnp.float32)

    def batch_rows(x):
        return jnp.reshape(
            jnp.broadcast_to(x[:, None, :], (nb, _M1, 128)), (rows_n, 128))

    for s in range(_NSEG):
        sl = slice(_STEP * s, _STEP * s + _NPERSEG)
        d = rows_ref[:, sl]
        y = jax.lax.dot_general(d, wc, dn, precision=hi,
                                preferred_element_type=jnp.float32)
        t = t_ref[:, 0, sl]
        x = jax.lax.dot_general(t, wc, dn, precision=hi,
                                preferred_element_type=jnp.float32)
        xs = jax.lax.dot_general(t, wsw, dn, precision=hi,
                                 preferred_element_type=jnp.float32)
        p2 = p2 + y * y
        m = m + batch_rows(x) * y
        n2 = n2 + batch_rows(xs) * y
        pxx2 = pxx2 + x * x
    fidx = jax.lax.broadcasted_iota(jnp.int32, (1, 128), 1)
    mid = jnp.logical_and(fidx >= 1, fidx <= 63)
    pyy = _fold(p2, mid)
    pr = _fold(m, mid)
    pi = _fold(n2, mid)
    pxx = batch_rows(_fold(pxx2, mid))
    cxy = (pr * pr + pi * pi) / (pxx * pyy + 1e-12)
    score = jnp.sum(jnp.where(fidx <= 64, cxy, 0.0), axis=1) * (1.0 / _NF)

    s2 = jnp.reshape(score, (nb, _M1))
    gidx = gidx_ref[...]
    cols = []
    for _ in range(_NREF):
        mx = jnp.max(s2, axis=1, keepdims=True)
        hit = s2 == mx
        g = jnp.min(jnp.where(hit, gidx, jnp.int32(2**30)), axis=1,
                    keepdims=True)
        cols.append(g)
        s2 = jnp.where(gidx == g, jnp.float32(-1.0), s2)
    out_ref[...] = jnp.concatenate(cols, axis=1)


def _sc_gather(db_flat, flat_idx, n_rows, L):
    info = plsc.get_sparse_core_info()
    nw = info.num_cores * info.num_subcores
    rows_per_w = n_rows // nw
    mesh = plsc.VectorSubcoreMesh(core_axis_name="c", subcore_axis_name="s")

    @functools.partial(
        pl.kernel, mesh=mesh,
        out_type=jax.ShapeDtypeStruct((n_rows, L), jnp.float32),
        scratch_types=[
            pltpu.VMEM((rows_per_w,), jnp.int32),
            pltpu.VMEM((rows_per_w, L), jnp.float32),
            pltpu.SemaphoreType.DMA,
        ],
    )
    def gather(db_hbm, idx_hbm, out_hbm, idx_v, rows_v, sem):
        wid = lax.axis_index("s") * info.num_cores + lax.axis_index("c")
        base = wid * rows_per_w
        pltpu.sync_copy(idx_hbm.at[pl.ds(base, rows_per_w)], idx_v)
        pltpu.async_copy(db_hbm.at[idx_v], rows_v, sem).wait()
        pltpu.sync_copy(rows_v, out_hbm.at[pl.ds(base, rows_per_w)])

    return gather(db_flat, flat_idx)


def kernel(target_series, TS_database):
    B, N, L = TS_database.shape
    nblk = (B * N) // _BLK
    wc, wsw = _dft_mats()
    wc_b, wsw_b = wc.astype(jnp.bfloat16), wsw.astype(jnp.bfloat16)
    db_flat = TS_database.reshape(B * N, L)
    blocks_per_batch = N // _BLK

    scores3 = pl.pallas_call(
        _stage1_kernel,
        grid=(nblk,),
        in_specs=[
            pl.BlockSpec((1, 1, L), lambda j: (j // blocks_per_batch, 0, 0)),
            pl.BlockSpec((_BLK, L), lambda j: (j, 0)),
            pl.BlockSpec((128, 128), lambda j: (0, 0)),
            pl.BlockSpec((128, 128), lambda j: (0, 0)),
        ],
        out_specs=pl.BlockSpec((1, _BLK, 1), lambda j: (j, 0, 0)),
        out_shape=jax.ShapeDtypeStruct((nblk, _BLK, 1), jnp.float32),
        compiler_params=pltpu.CompilerParams(
            dimension_semantics=("arbitrary",)),
    )(target_series, db_flat, wc_b, wsw_b)
    scores = scores3.reshape(B, N)

    cand_idx = pl.pallas_call(
        _topk_kernel,
        out_shape=jax.ShapeDtypeStruct((B, _M1), jnp.int32),
    )(scores)

    cand_rows = _sc_gather(db_flat, cand_idx.reshape(B * _M1), B * _M1, L)

    topk_idx = pl.pallas_call(
        _rescore_kernel,
        out_shape=jax.ShapeDtypeStruct((B, _NREF), jnp.int32),
    )(target_series, cand_rows, cand_idx, wc, wsw)

    gathered = _sc_gather(db_flat, topk_idx.reshape(B * _NREF), B * _NREF, L)
    return gathered.reshape(B, _NREF, L)
